# baseline (speedup 1.0000x reference)
import jax
import jax.numpy as jnp
from jax.experimental import pallas as pl
from jax.experimental.pallas import tpu as pltpu

_CHUNKS = 8


def _dma_body(x_ref, o_ref, sems):
    rows = x_ref.shape[0]
    per = rows // _CHUNKS
    copies = [
        pltpu.make_async_copy(
            x_ref.at[pl.ds(k * per, per), :],
            o_ref.at[pl.ds(k * per, per), :],
            sems.at[k],
        )
        for k in range(_CHUNKS)
    ]
    for c in copies:
        c.start()
    for c in copies:
        c.wait()


def kernel(images, features, proposals, gt_bboxes, gt_labels):
    n, c, h, w = images.shape
    x = images.reshape(n * c * h, w)
    out = pl.pallas_call(
        _dma_body,
        out_shape=jax.ShapeDtypeStruct(x.shape, x.dtype),
        in_specs=[pl.BlockSpec(memory_space=pl.ANY)],
        out_specs=pl.BlockSpec(memory_space=pl.ANY),
        scratch_shapes=[pltpu.SemaphoreType.DMA((_CHUNKS,))],
    )(x)
    return out.reshape(images.shape)


# speedup vs baseline: 12.2993x; 12.2993x over previous
import jax
import jax.numpy as jnp
from jax.experimental import pallas as pl
from jax.experimental.pallas import tpu as pltpu


def _copy_body(x_ref, o_ref):
    o_ref[...] = x_ref[...]


def kernel(images, features, proposals, gt_bboxes, gt_labels):
    n, c, h, w = images.shape
    x = images.reshape(n * c * h, w)
    rows = x.shape[0]
    grid = 24
    block_rows = rows // grid
    out = pl.pallas_call(
        _copy_body,
        out_shape=jax.ShapeDtypeStruct(x.shape, x.dtype),
        grid=(grid,),
        in_specs=[pl.BlockSpec((block_rows, w), lambda i: (i, 0))],
        out_specs=pl.BlockSpec((block_rows, w), lambda i: (i, 0)),
        compiler_params=pltpu.CompilerParams(
            dimension_semantics=("parallel",),
        ),
    )(x)
    return out.reshape(images.shape)


# speedup vs baseline: 27.3170x; 2.2210x over previous
import jax
import jax.numpy as jnp
from jax.experimental import pallas as pl
from jax.experimental.pallas import tpu as pltpu


def _copy_body(x_ref, o_ref):
    o_ref[...] = x_ref[...]


def kernel(images, features, proposals, gt_bboxes, gt_labels):
    n, c, h, w = images.shape
    x = images.reshape(n * c * h, w)
    rows = x.shape[0]
    grid = 6
    block_rows = rows // grid
    out = pl.pallas_call(
        _copy_body,
        out_shape=jax.ShapeDtypeStruct(x.shape, x.dtype),
        grid=(grid,),
        in_specs=[pl.BlockSpec((block_rows, w), lambda i: (i, 0))],
        out_specs=pl.BlockSpec((block_rows, w), lambda i: (i, 0)),
        compiler_params=pltpu.CompilerParams(
            dimension_semantics=("parallel",),
        ),
    )(x)
    return out.reshape(images.shape)


# speedup vs baseline: 30.7542x; 1.1258x over previous
import jax
import jax.numpy as jnp
from jax.experimental import pallas as pl
from jax.experimental.pallas import tpu as pltpu


def _copy_body(x_ref, o_ref):
    o_ref[...] = x_ref[...]


def kernel(images, features, proposals, gt_bboxes, gt_labels):
    n, c, h, w = images.shape
    x = images.reshape(n * c * h, w)
    rows = x.shape[0]
    grid = 3
    block_rows = rows // grid
    out = pl.pallas_call(
        _copy_body,
        out_shape=jax.ShapeDtypeStruct(x.shape, x.dtype),
        grid=(grid,),
        in_specs=[pl.BlockSpec((block_rows, w), lambda i: (i, 0))],
        out_specs=pl.BlockSpec((block_rows, w), lambda i: (i, 0)),
        compiler_params=pltpu.CompilerParams(
            dimension_semantics=("parallel",),
        ),
    )(x)
    return out.reshape(images.shape)


# speedup vs baseline: 40.0272x; 1.3015x over previous
import jax
import jax.numpy as jnp
from jax.experimental import pallas as pl
from jax.experimental.pallas import tpu as pltpu


def _copy_body(x_ref, o_ref):
    o_ref[...] = x_ref[...]


def kernel(images, features, proposals, gt_bboxes, gt_labels):
    n, c, h, w = images.shape
    x = images.reshape(n * c * h, w)
    rows = x.shape[0]
    grid = 2
    block_rows = rows // grid
    out = pl.pallas_call(
        _copy_body,
        out_shape=jax.ShapeDtypeStruct(x.shape, x.dtype),
        grid=(grid,),
        in_specs=[pl.BlockSpec((block_rows, w), lambda i: (i, 0))],
        out_specs=pl.BlockSpec((block_rows, w), lambda i: (i, 0)),
        compiler_params=pltpu.CompilerParams(
            dimension_semantics=("parallel",),
        ),
    )(x)
    return out.reshape(images.shape)
